# baseline (device time: 228313 ns/iter reference)
import jax
import jax.numpy as jnp
from jax import lax
from jax.experimental import pallas as pl
from jax.experimental.pallas import tpu as pltpu

M, N = 16384, 1024
DTOT = 3072
DH = 1536
DC = 512
STOT = 2560
SC = 640
SH = 1280
SIG = DTOT + STOT
SBASE = 2 * DTOT
RBASE = DH + STOT
YIN = DH + STOT + SH


def kernel(x):
    def body(x_hbm, out_hbm, mine_bf, recv_xb, recv_yb, recv_zb, stage,
             load_sems, send_x, recv_sx,
             send_yD, send_yS, send_yR, recv_yD, recv_yS, recv_yR,
             send_zD, send_zS, send_zR, recv_zD, recv_zS, recv_zR,
             out_sx, out_y, out_z):
        my_x = lax.axis_index("x")
        my_y = lax.axis_index("y")
        my_z = lax.axis_index("z")
        py = lax.rem(my_y, 2)
        pz = lax.rem(my_z, 2)
        q = py ^ pz
        i = py * 2 + pz
        partner = (1 - my_x, my_y, my_z)
        ynbr = (my_x, my_y ^ 1, my_z)
        znbr = (my_x, my_y, my_z ^ 1)

        dq0 = q * DTOT
        do0 = (1 - q) * DTOT
        s0 = SBASE + i * STOT
        s_y0 = SBASE + (i ^ 2) * STOT
        s_z0 = SBASE + (i ^ 1) * STOT
        s_d0 = SBASE + (i ^ 3) * STOT

        xchunks = (
            [(dq0 + j * DC, j * DC, DC) for j in range(6)]
            + [(s0 + k * SC, DTOT + k * SC, SC) for k in range(4)]
        )

        def load(c, slot):
            g, _, r = xchunks[c]
            return pltpu.make_async_copy(
                x_hbm.at[pl.ds(g, r), :], stage.at[slot, 0:r], load_sems.at[slot])

        def wait_dma(sem, slc):
            pltpu.make_async_copy(slc, slc, sem).wait()

        load(0, 0).start()

        bsem = pltpu.get_barrier_semaphore()
        for nbr in (partner, ynbr, znbr):
            pl.semaphore_signal(bsem, inc=1, device_id=nbr,
                                device_id_type=pl.DeviceIdType.MESH)
        pl.semaphore_wait(bsem, 3)

        for c in range(10):
            g, l, r = xchunks[c]
            if c + 1 < 10:
                load(c + 1, (c + 1) % 2).start()
            load(c, c % 2).wait()
            mine_bf[pl.ds(l, r), :] = stage[c % 2, 0:r].astype(jnp.bfloat16)
            pltpu.make_async_remote_copy(
                src_ref=mine_bf.at[pl.ds(l, r), :],
                dst_ref=recv_xb.at[pl.ds(l, r), :],
                send_sem=send_x.at[c], recv_sem=recv_sx.at[c],
                device_id=partner, device_id_type=pl.DeviceIdType.MESH,
            ).start()

        def rsend(src, dst, ssem, rsem, dev):
            pltpu.make_async_remote_copy(
                src_ref=src, dst_ref=dst, send_sem=ssem, recv_sem=rsem,
                device_id=dev, device_id_type=pl.DeviceIdType.MESH).start()

        def store_out(src, gl, r, sem):
            pltpu.make_async_copy(src, out_hbm.at[pl.ds(gl, r), :], sem).start()

        for j in range(6):
            g, l, r = xchunks[j]
            rows = pl.ds(l, r)
            wait_dma(recv_sx.at[j], recv_xb.at[rows])
            recv_xb[rows, :] = recv_xb[rows, :] + mine_bf[rows, :]
            if j < 3:
                rsend(recv_xb.at[rows], recv_yb.at[rows],
                      send_yD.at[j], recv_yD.at[j], ynbr)
            else:
                rsend(recv_xb.at[rows], recv_zb.at[pl.ds((j - 3) * DC, DC)],
                      send_zD.at[j - 3], recv_zD.at[j - 3], znbr)
            store_out(recv_xb.at[rows], g, r, out_sx.at[j])

        for k in range(4):
            g, l, r = xchunks[6 + k]
            rows = pl.ds(l, r)
            wait_dma(recv_sx.at[6 + k], recv_xb.at[rows])
            recv_xb[rows, :] = recv_xb[rows, :] + mine_bf[rows, :]
            nrows = pl.ds(DH + k * SC, SC)
            rsend(recv_xb.at[rows], recv_yb.at[nrows],
                  send_yS.at[k], recv_yS.at[k], ynbr)
            rsend(recv_xb.at[rows], recv_zb.at[nrows],
                  send_zS.at[k], recv_zS.at[k], znbr)
            store_out(recv_xb.at[rows], g, r, out_sx.at[6 + k])
            if k < 2:
                zrows = pl.ds(DH + k * SC, SC)
                wait_dma(recv_zS.at[k], recv_zb.at[zrows])
                rsend(recv_zb.at[zrows], recv_yb.at[pl.ds(RBASE + k * SC, SC)],
                      send_yR.at[k], recv_yR.at[k], ynbr)
                store_out(recv_zb.at[zrows], s_z0 + k * SC, SC, out_z.at[3 + k])
            else:
                yrows = pl.ds(DH + k * SC, SC)
                wait_dma(recv_yS.at[k], recv_yb.at[yrows])
                rsend(recv_yb.at[yrows], recv_zb.at[pl.ds(RBASE + (k - 2) * SC, SC)],
                      send_zR.at[k - 2], recv_zR.at[k - 2], znbr)
                store_out(recv_yb.at[yrows], s_y0 + k * SC, SC, out_y.at[3 + k])

        for k in range(2, 4):
            zrows = pl.ds(DH + k * SC, SC)
            wait_dma(recv_zS.at[k], recv_zb.at[zrows])
            store_out(recv_zb.at[zrows], s_z0 + k * SC, SC, out_z.at[3 + k])
        for k in range(0, 2):
            yrows = pl.ds(DH + k * SC, SC)
            wait_dma(recv_yS.at[k], recv_yb.at[yrows])
            store_out(recv_yb.at[yrows], s_y0 + k * SC, SC, out_y.at[3 + k])
        for j in range(3):
            rows = pl.ds(j * DC, DC)
            wait_dma(recv_yD.at[j], recv_yb.at[rows])
            store_out(recv_yb.at[rows], do0 + j * DC, DC, out_y.at[j])
        for j in range(3):
            rows = pl.ds(j * DC, DC)
            wait_dma(recv_zD.at[j], recv_zb.at[rows])
            store_out(recv_zb.at[rows], do0 + DH + j * DC, DC, out_z.at[j])
        for k in range(2):
            rows = pl.ds(RBASE + k * SC, SC)
            wait_dma(recv_yR.at[k], recv_yb.at[rows])
            store_out(recv_yb.at[rows], s_d0 + k * SC, SC, out_y.at[7 + k])
        for k in range(2):
            rows = pl.ds(RBASE + k * SC, SC)
            wait_dma(recv_zR.at[k], recv_zb.at[rows])
            store_out(recv_zb.at[rows], s_d0 + SH + k * SC, SC, out_z.at[7 + k])

        for c in range(10):
            g, l, r = xchunks[c]
            wait_dma(send_x.at[c], mine_bf.at[pl.ds(l, r), :])
            wait_dma(out_sx.at[c], recv_xb.at[pl.ds(l, r), :])
        for j in range(3):
            wait_dma(send_yD.at[j], recv_xb.at[pl.ds(j * DC, DC), :])
            wait_dma(send_zD.at[j], recv_xb.at[pl.ds(DH + j * DC, DC), :])
            wait_dma(out_y.at[j], recv_yb.at[pl.ds(j * DC, DC), :])
            wait_dma(out_z.at[j], recv_zb.at[pl.ds(j * DC, DC), :])
        for k in range(4):
            srows = pl.ds(DTOT + k * SC, SC)
            wait_dma(send_yS.at[k], recv_xb.at[srows])
            wait_dma(send_zS.at[k], recv_xb.at[srows])
            wait_dma(out_y.at[3 + k], recv_yb.at[pl.ds(DH + k * SC, SC)])
            wait_dma(out_z.at[3 + k], recv_zb.at[pl.ds(DH + k * SC, SC)])
        for k in range(2):
            wait_dma(send_yR.at[k], recv_zb.at[pl.ds(DH + k * SC, SC)])
            wait_dma(send_zR.at[k], recv_yb.at[pl.ds(DH + (2 + k) * SC, SC)])
            wait_dma(out_y.at[7 + k], recv_yb.at[pl.ds(RBASE + k * SC, SC)])
            wait_dma(out_z.at[7 + k], recv_zb.at[pl.ds(RBASE + k * SC, SC)])

    return pl.pallas_call(
        body,
        out_shape=jax.ShapeDtypeStruct((M, N), jnp.bfloat16),
        in_specs=[pl.BlockSpec(memory_space=pltpu.MemorySpace.HBM)],
        out_specs=pl.BlockSpec(memory_space=pltpu.MemorySpace.HBM),
        scratch_shapes=[
            pltpu.VMEM((SIG, N), jnp.bfloat16),
            pltpu.VMEM((SIG, N), jnp.bfloat16),
            pltpu.VMEM((YIN, N), jnp.bfloat16),
            pltpu.VMEM((YIN, N), jnp.bfloat16),
            pltpu.VMEM((2, SC, N), jnp.float32),
            pltpu.SemaphoreType.DMA((2,)),
            pltpu.SemaphoreType.DMA((10,)),
            pltpu.SemaphoreType.DMA((10,)),
            pltpu.SemaphoreType.DMA((3,)),
            pltpu.SemaphoreType.DMA((4,)),
            pltpu.SemaphoreType.DMA((2,)),
            pltpu.SemaphoreType.DMA((3,)),
            pltpu.SemaphoreType.DMA((4,)),
            pltpu.SemaphoreType.DMA((2,)),
            pltpu.SemaphoreType.DMA((3,)),
            pltpu.SemaphoreType.DMA((4,)),
            pltpu.SemaphoreType.DMA((2,)),
            pltpu.SemaphoreType.DMA((3,)),
            pltpu.SemaphoreType.DMA((4,)),
            pltpu.SemaphoreType.DMA((2,)),
            pltpu.SemaphoreType.DMA((10,)),
            pltpu.SemaphoreType.DMA((9,)),
            pltpu.SemaphoreType.DMA((9,)),
        ],
        compiler_params=pltpu.CompilerParams(
            collective_id=0, vmem_limit_bytes=64 * 1024 * 1024),
    )(x)


# device time: 222564 ns/iter; 1.0258x vs baseline; 1.0258x over previous
import jax
import jax.numpy as jnp
from jax import lax
from jax.experimental import pallas as pl
from jax.experimental.pallas import tpu as pltpu

M, N = 16384, 1024
HALF = M // 2
C = 16
R = HALF // C


def kernel(x):
    def body(x_hbm, out_hbm, mine_bf, recv_x, recv_y, stage,
             load_sems, out_sems, out_sems2, send_x, recv_sx, send_y, recv_sy):
        my_x = lax.axis_index("x")
        my_y = lax.axis_index("y")
        my_z = lax.axis_index("z")
        p = lax.rem(my_y, 2)
        row0 = p * HALF
        partner = (1 - my_x, my_y, my_z)
        ynbr = (my_x, my_y ^ 1, my_z)

        def load(c, slot):
            return pltpu.make_async_copy(
                x_hbm.at[pl.ds(row0 + c * R, R), :],
                stage.at[slot], load_sems.at[slot])

        load(0, 0).start()

        bsem = pltpu.get_barrier_semaphore()
        for nbr in (partner, ynbr):
            pl.semaphore_signal(bsem, inc=1, device_id=nbr,
                                device_id_type=pl.DeviceIdType.MESH)
        pl.semaphore_wait(bsem, 2)

        for c in range(C):
            if c + 1 < C:
                load(c + 1, (c + 1) % 2).start()
            load(c, c % 2).wait()
            mine_bf[pl.ds(c * R, R), :] = stage[c % 2].astype(jnp.bfloat16)
            pltpu.make_async_remote_copy(
                src_ref=mine_bf.at[pl.ds(c * R, R), :],
                dst_ref=recv_x.at[pl.ds(c * R, R), :],
                send_sem=send_x.at[c], recv_sem=recv_sx.at[c],
                device_id=partner, device_id_type=pl.DeviceIdType.MESH,
            ).start()

        for c in range(C):
            rows = pl.ds(c * R, R)
            out_rows = pl.ds(row0 + c * R, R)
            pltpu.make_async_copy(recv_x.at[rows], recv_x.at[rows],
                                  recv_sx.at[c]).wait()
            recv_x[rows, :] = recv_x[rows, :] + mine_bf[rows, :]
            pltpu.make_async_remote_copy(
                src_ref=recv_x.at[rows],
                dst_ref=recv_y.at[rows],
                send_sem=send_y.at[c], recv_sem=recv_sy.at[c],
                device_id=ynbr, device_id_type=pl.DeviceIdType.MESH,
            ).start()
            pltpu.make_async_copy(
                recv_x.at[rows], out_hbm.at[out_rows, :], out_sems.at[c]
            ).start()

        other0 = (1 - p) * HALF
        for c in range(C):
            rows = pl.ds(c * R, R)
            out_rows = pl.ds(row0 + c * R, R)
            oth_rows = pl.ds(other0 + c * R, R)
            pltpu.make_async_copy(recv_x.at[rows], recv_y.at[rows],
                                  recv_sy.at[c]).wait()
            pltpu.make_async_copy(
                recv_y.at[rows], out_hbm.at[oth_rows, :], out_sems2.at[c]
            ).start()
        for c in range(C):
            rows = pl.ds(c * R, R)
            out_rows = pl.ds(row0 + c * R, R)
            oth_rows = pl.ds(other0 + c * R, R)
            pltpu.make_async_copy(recv_y.at[rows], out_hbm.at[oth_rows, :],
                                  out_sems2.at[c]).wait()
            pltpu.make_async_copy(recv_x.at[rows], out_hbm.at[out_rows, :],
                                  out_sems.at[c]).wait()
            pltpu.make_async_copy(mine_bf.at[rows], recv_x.at[rows],
                                  send_x.at[c]).wait()
            pltpu.make_async_copy(recv_x.at[rows], recv_y.at[rows],
                                  send_y.at[c]).wait()

    return pl.pallas_call(
        body,
        out_shape=jax.ShapeDtypeStruct((M, N), jnp.bfloat16),
        in_specs=[pl.BlockSpec(memory_space=pltpu.MemorySpace.HBM)],
        out_specs=pl.BlockSpec(memory_space=pltpu.MemorySpace.HBM),
        scratch_shapes=[
            pltpu.VMEM((HALF, N), jnp.bfloat16),
            pltpu.VMEM((HALF, N), jnp.bfloat16),
            pltpu.VMEM((HALF, N), jnp.bfloat16),
            pltpu.VMEM((2, R, N), jnp.float32),
            pltpu.SemaphoreType.DMA((2,)),
            pltpu.SemaphoreType.DMA((C,)),
            pltpu.SemaphoreType.DMA((C,)),
            pltpu.SemaphoreType.DMA((C,)),
            pltpu.SemaphoreType.DMA((C,)),
            pltpu.SemaphoreType.DMA((C,)),
            pltpu.SemaphoreType.DMA((C,)),
        ],
        compiler_params=pltpu.CompilerParams(
            collective_id=0, vmem_limit_bytes=64 * 1024 * 1024),
    )(x)
